# baseline (speedup 1.0000x reference)
import functools

import jax
import jax.numpy as jnp
from jax import lax
from jax.experimental import pallas as pl
from jax.experimental.pallas import tpu as pltpu
from jax.experimental.pallas import tpu_sc as plsc

BATCH = 4096
SEQ = 200
D = 64
B_TOTAL = BATCH * SEQ
NC = 2
NS_SUB = 16
NW = NC * NS_SUB
B_PER_W = B_TOTAL // NW
CHUNK = 128
NCHUNK = B_PER_W // CHUNK
K = 4
NSUP = NCHUNK // K

_mesh = plsc.VectorSubcoreMesh(core_axis_name="c", subcore_axis_name="s")


@functools.partial(
    pl.kernel,
    mesh=_mesh,
    out_type=jax.ShapeDtypeStruct((B_TOTAL // CHUNK, CHUNK, D), jnp.float32),
    scratch_types=[
        pltpu.VMEM((NCHUNK, CHUNK), jnp.int32),
        pltpu.VMEM((K, CHUNK, D), jnp.float32),
        pltpu.VMEM((K, CHUNK, D), jnp.float32),
        pltpu.SemaphoreType.DMA,
        pltpu.SemaphoreType.DMA,
    ],
    compiler_params=pltpu.CompilerParams(use_tc_tiling_on_sc=False),
)
def _emb_lookup(x_hbm, table_hbm, out_hbm, idx_v, rows0, rows1, gsem0, gsem1):
    wid = lax.axis_index("s") * NC + lax.axis_index("c")
    chunk_base = wid * NCHUNK

    def fire(s, rows, gsem):
        for j in range(K):
            pltpu.make_async_copy(
                table_hbm.at[idx_v.at[s * K + j]], rows.at[j], gsem
            ).start()

    def drain(rows, gsem):
        for j in range(K):
            pltpu.make_async_copy(
                table_hbm.at[idx_v.at[0]], rows.at[j], gsem
            ).wait()

    def write_out(s, rows):
        pltpu.sync_copy(rows, out_hbm.at[pl.ds(chunk_base + s * K, K)])

    pltpu.sync_copy(x_hbm.at[wid], idx_v)
    fire(0, rows0, gsem0)

    def body(g2, carry):
        s0 = g2 * 2
        s1 = s0 + 1
        fire(s1, rows1, gsem1)
        drain(rows0, gsem0)
        write_out(s0, rows0)
        fire(s1 + 1, rows0, gsem0)
        drain(rows1, gsem1)
        write_out(s1, rows1)
        return carry

    lax.fori_loop(0, NSUP // 2 - 1, body, 0)

    fire(NSUP - 1, rows1, gsem1)
    drain(rows0, gsem0)
    write_out(NSUP - 2, rows0)
    drain(rows1, gsem1)
    write_out(NSUP - 1, rows1)


def kernel(x, table):
    xr = x.astype(jnp.int32).reshape(NW, NCHUNK, CHUNK)
    out = _emb_lookup(xr, table)
    return out.reshape(BATCH, SEQ, D)


# speedup vs baseline: 1.0009x; 1.0009x over previous
import functools

import jax
import jax.numpy as jnp
from jax import lax
from jax.experimental import pallas as pl
from jax.experimental.pallas import tpu as pltpu
from jax.experimental.pallas import tpu_sc as plsc

BATCH = 4096
SEQ = 200
D = 64
NC = 2
NS_SUB = 16
NW = NC * NS_SUB
ROWS_PER_W = BATCH // NW
SPLIT = 128
REST = SEQ - SPLIT
R = 2
NSUP = ROWS_PER_W // R

_mesh = plsc.VectorSubcoreMesh(core_axis_name="c", subcore_axis_name="s")


@functools.partial(
    pl.kernel,
    mesh=_mesh,
    out_type=jax.ShapeDtypeStruct((BATCH, SEQ, D), jnp.float32),
    scratch_types=[
        pltpu.VMEM((ROWS_PER_W, SEQ), jnp.int32),
        pltpu.VMEM((R, SEQ, D), jnp.float32),
        pltpu.VMEM((R, SEQ, D), jnp.float32),
        pltpu.SemaphoreType.DMA,
        pltpu.SemaphoreType.DMA,
    ],
    compiler_params=pltpu.CompilerParams(use_tc_tiling_on_sc=False),
)
def _emb_lookup(x_hbm, table_hbm, out_hbm, idx_v, rows0, rows1, gsem0, gsem1):
    wid = lax.axis_index("s") * NC + lax.axis_index("c")
    row_base = wid * ROWS_PER_W

    def fire(s, rows, gsem):
        for rr in range(R):
            row = s * R + rr
            pltpu.make_async_copy(
                table_hbm.at[idx_v.at[row, pl.ds(0, SPLIT)]],
                rows.at[rr, pl.ds(0, SPLIT)], gsem,
            ).start()
            pltpu.make_async_copy(
                table_hbm.at[idx_v.at[row, pl.ds(SPLIT, REST)]],
                rows.at[rr, pl.ds(SPLIT, REST)], gsem,
            ).start()

    def drain(rows, gsem):
        for rr in range(R):
            pltpu.make_async_copy(
                table_hbm.at[idx_v.at[0, pl.ds(0, SPLIT)]],
                rows.at[rr, pl.ds(0, SPLIT)], gsem,
            ).wait()
            pltpu.make_async_copy(
                table_hbm.at[idx_v.at[0, pl.ds(SPLIT, REST)]],
                rows.at[rr, pl.ds(SPLIT, REST)], gsem,
            ).wait()

    def write_out(s, rows):
        pltpu.sync_copy(rows, out_hbm.at[pl.ds(row_base + s * R, R)])

    pltpu.sync_copy(x_hbm.at[pl.ds(row_base, ROWS_PER_W)], idx_v)
    fire(0, rows0, gsem0)

    def body(g2, carry):
        s0 = g2 * 2
        s1 = s0 + 1
        fire(s1, rows1, gsem1)
        drain(rows0, gsem0)
        write_out(s0, rows0)
        fire(s1 + 1, rows0, gsem0)
        drain(rows1, gsem1)
        write_out(s1, rows1)
        return carry

    lax.fori_loop(0, NSUP // 2 - 1, body, 0)

    fire(NSUP - 1, rows1, gsem1)
    drain(rows0, gsem0)
    write_out(NSUP - 2, rows0)
    drain(rows1, gsem1)
    write_out(NSUP - 1, rows1)


def kernel(x, table):
    return _emb_lookup(x.astype(jnp.int32), table)


# speedup vs baseline: 1.2193x; 1.2182x over previous
import functools

import jax
import jax.numpy as jnp
from jax import lax
from jax.experimental import pallas as pl
from jax.experimental.pallas import tpu as pltpu
from jax.experimental.pallas import tpu_sc as plsc

BATCH = 4096
SEQ = 200
D = 64
DPAD = 128
VOCAB_ROWS = 1000000
NC = 2
NS_SUB = 16
NW = NC * NS_SUB
ROWS_PER_W = BATCH // NW
SPLIT = 128
REST = SEQ - SPLIT

_mesh = plsc.VectorSubcoreMesh(core_axis_name="c", subcore_axis_name="s")

@functools.partial(
    pl.kernel,
    mesh=_mesh,
    out_type=jax.ShapeDtypeStruct((BATCH, SEQ, DPAD), jnp.float32),
    scratch_types=[
        pltpu.VMEM((ROWS_PER_W, SEQ), jnp.int32),
        pltpu.VMEM((SEQ, DPAD), jnp.float32),
        pltpu.VMEM((SEQ, DPAD), jnp.float32),
        pltpu.SemaphoreType.DMA,
        pltpu.SemaphoreType.DMA,
    ],
)
def _emb_lookup(x_hbm, tpad_hbm, out_hbm, idx_v, rows0, rows1, gsem0, gsem1):
    wid = lax.axis_index("s") * NC + lax.axis_index("c")
    row_base = wid * ROWS_PER_W

    def fire(r, rows, gsem):
        pltpu.make_async_copy(
            tpad_hbm.at[idx_v.at[r, pl.ds(0, SPLIT)]],
            rows.at[pl.ds(0, SPLIT)], gsem,
        ).start()
        pltpu.make_async_copy(
            tpad_hbm.at[idx_v.at[r, pl.ds(SPLIT, REST)]],
            rows.at[pl.ds(SPLIT, REST)], gsem,
        ).start()

    def drain(rows, gsem):
        pltpu.make_async_copy(
            tpad_hbm.at[idx_v.at[0, pl.ds(0, SPLIT)]],
            rows.at[pl.ds(0, SPLIT)], gsem,
        ).wait()
        pltpu.make_async_copy(
            tpad_hbm.at[idx_v.at[0, pl.ds(SPLIT, REST)]],
            rows.at[pl.ds(SPLIT, REST)], gsem,
        ).wait()

    def write_out(r, rows):
        pltpu.sync_copy(rows, out_hbm.at[row_base + r])

    pltpu.sync_copy(x_hbm.at[pl.ds(row_base, ROWS_PER_W)], idx_v)
    fire(0, rows0, gsem0)

    def body(g2, carry):
        r0 = g2 * 2
        r1 = r0 + 1
        fire(r1, rows1, gsem1)
        drain(rows0, gsem0)
        write_out(r0, rows0)
        fire(r1 + 1, rows0, gsem0)
        drain(rows1, gsem1)
        write_out(r1, rows1)
        return carry

    lax.fori_loop(0, ROWS_PER_W // 2 - 1, body, 0)

    fire(ROWS_PER_W - 1, rows1, gsem1)
    drain(rows0, gsem0)
    write_out(ROWS_PER_W - 2, rows0)
    drain(rows1, gsem1)
    write_out(ROWS_PER_W - 1, rows1)


def kernel(x, table):
    tpad = jnp.pad(table, ((0, 0), (0, DPAD - D)))
    out = _emb_lookup(x.astype(jnp.int32), tpad)
    return out[:, :, :D]
